# baseline (speedup 1.0000x reference)
import functools

import jax
import jax.numpy as jnp
from jax import lax
from jax.experimental import pallas as pl
from jax.experimental.pallas import tpu as pltpu
from jax.experimental.pallas import tpu_sc as plsc


@functools.lru_cache(maxsize=None)
def _make_sc_permute(R, SL, B):
    info = plsc.get_sparse_core_info()
    NC, NS = info.num_cores, info.num_subcores
    NW = NC * NS
    assert R % NW == 0, (R, NW)
    spw = R // NW
    assert spw % 2 == 0

    mesh = plsc.VectorSubcoreMesh(core_axis_name="c", subcore_axis_name="s")

    @functools.partial(
        pl.kernel,
        out_type=jax.ShapeDtypeStruct((R, SL, B), jnp.float32),
        mesh=mesh,
        scratch_types=[
            pltpu.VMEM((spw, 1), jnp.int32),
            pltpu.VMEM((1, SL, B), jnp.float32),
            pltpu.VMEM((1, SL, B), jnp.float32),
            pltpu.VMEM((1, SL, B), jnp.float32),
            pltpu.SemaphoreType.DMA,
            pltpu.SemaphoreType.DMA,
        ],
    )
    def k(xs_hbm, sidx_hbm, out_hbm, idx_t, buf_a, buf_b, buf_c, sem_g, sem_s):
        wid = lax.axis_index("s") * NC + lax.axis_index("c")
        pltpu.sync_copy(sidx_hbm.at[wid], idx_t)
        bufs = (buf_a, buf_b, buf_c)
        for b in range(2):
            pltpu.async_copy(xs_hbm.at[idx_t.at[b]], bufs[b], sem_g)

        def step(kk, b):
            pltpu.make_async_copy(xs_hbm.at[pl.ds(0, 1)], bufs[b], sem_g).wait()
            pltpu.async_copy(bufs[b], out_hbm.at[pl.ds(wid * spw + kk, 1)], sem_s)

            @pl.when(kk >= 1)
            def _():
                pltpu.make_async_copy(bufs[b], out_hbm.at[pl.ds(0, 1)], sem_s).wait()

            @pl.when(kk + 2 < spw)
            def _():
                pltpu.async_copy(
                    xs_hbm.at[idx_t.at[kk + 2]], bufs[(b + 2) % 3], sem_g
                )

        def body(it, carry):
            for b in range(3):
                step(it * 3 + b, b)
            return carry

        n3 = (spw // 3) * 3
        lax.fori_loop(0, spw // 3, body, 0)
        for kk in range(n3, spw):
            step(kk, kk % 3)
        pltpu.make_async_copy(buf_a, out_hbm.at[pl.ds(0, 1)], sem_s).wait()

    return k


def kernel(x, order):
    B, S, D = x.shape
    SL = 8
    NSTR = D // SL
    R = S * NSTR
    xs = jnp.transpose(x, (1, 2, 0)).reshape(R, SL, B)
    spw = R // 32
    w_i = lax.broadcasted_iota(jnp.int32, (32, spw, 1), 0)
    k_i = lax.broadcasted_iota(jnp.int32, (32, spw, 1), 1)
    j = w_i * spw + k_i
    slab = j // NSTR
    sidx = jnp.take(order, slab) * NSTR + (j - slab * NSTR)
    out8 = _make_sc_permute(R, SL, B)(xs, sidx)
    out_t = out8.reshape(S, D, B)
    return jnp.transpose(out_t, (2, 0, 1))


# speedup vs baseline: 1.0595x; 1.0595x over previous
import functools

import jax
import jax.numpy as jnp
from jax import lax
from jax.experimental import pallas as pl
from jax.experimental.pallas import tpu as pltpu
from jax.experimental.pallas import tpu_sc as plsc


@functools.lru_cache(maxsize=None)
def _make_sc_permute(R, SL, B):
    info = plsc.get_sparse_core_info()
    NC, NS = info.num_cores, info.num_subcores
    NW = NC * NS
    assert R % NW == 0, (R, NW)
    spw = R // NW
    assert spw % 2 == 0

    mesh = plsc.VectorSubcoreMesh(core_axis_name="c", subcore_axis_name="s")

    @functools.partial(
        pl.kernel,
        out_type=jax.ShapeDtypeStruct((R, SL, B), jnp.float32),
        mesh=mesh,
        scratch_types=[
            pltpu.VMEM((spw, 1), jnp.int32),
            pltpu.VMEM((1, SL, B), jnp.float32),
            pltpu.VMEM((1, SL, B), jnp.float32),
            pltpu.VMEM((1, SL, B), jnp.float32),
            pltpu.SemaphoreType.DMA,
            pltpu.SemaphoreType.DMA,
        ],
    )
    def k(xs_hbm, sidx_hbm, out_hbm, idx_t, buf_a, buf_b, buf_c, sem_g, sem_s):
        wid = lax.axis_index("s") * NC + lax.axis_index("c")
        pltpu.sync_copy(sidx_hbm.at[wid], idx_t)
        bufs = (buf_a, buf_b, buf_c)
        for b in range(2):
            pltpu.async_copy(xs_hbm.at[idx_t.at[b]], bufs[b], sem_g)

        def step(kk, b):
            pltpu.make_async_copy(xs_hbm.at[pl.ds(0, 1)], bufs[b], sem_g).wait()
            pltpu.async_copy(bufs[b], out_hbm.at[pl.ds(wid * spw + kk, 1)], sem_s)

            @pl.when(kk >= 1)
            def _():
                pltpu.make_async_copy(bufs[b], out_hbm.at[pl.ds(0, 1)], sem_s).wait()

            @pl.when(kk + 2 < spw)
            def _():
                pltpu.async_copy(
                    xs_hbm.at[idx_t.at[kk + 2]], bufs[(b + 2) % 3], sem_g
                )

        def body(it, carry):
            for b in range(3):
                step(it * 3 + b, b)
            return carry

        n3 = (spw // 3) * 3
        lax.fori_loop(0, spw // 3, body, 0)
        for kk in range(n3, spw):
            step(kk, kk % 3)
        pltpu.make_async_copy(buf_a, out_hbm.at[pl.ds(0, 1)], sem_s).wait()

    return k


def kernel(x, order):
    B, S, D = x.shape
    SL = 8
    NSTR = D // SL
    R = S * NSTR
    xs = jnp.transpose(x, (1, 2, 0)).reshape(R, SL, B)
    sidx = (
        jnp.repeat(order * NSTR, NSTR)
        + jnp.tile(jnp.arange(NSTR, dtype=order.dtype), S)
    ).reshape(32, R // 32, 1)
    out8 = _make_sc_permute(R, SL, B)(xs, sidx)
    out_t = out8.reshape(S, D, B)
    return jnp.transpose(out_t, (2, 0, 1))
